# Initial kernel scaffold; baseline (speedup 1.0000x reference)
import functools

import jax
import jax.numpy as jnp
from jax import lax
from jax.experimental import pallas as pl
from jax.experimental.pallas import tpu as pltpu
from jax.experimental.pallas import tpu_sc as plsc

_NC = 2
_NS = 16
_NW = _NC * _NS
_K = 128
_BM = 400


def _sc_mesh():
    return plsc.VectorSubcoreMesh(core_axis_name="c", subcore_axis_name="s")


def _make_sc_deg(n_pad, chunks):
    zchunks = n_pad // _NS // _K

    @functools.partial(
        pl.kernel,
        mesh=_sc_mesh(),
        out_type=jax.ShapeDtypeStruct((_NC, n_pad), jnp.float32),
        scratch_types=[
            pltpu.VMEM((chunks, _K), jnp.int32),
            pltpu.VMEM((_K,), jnp.float32),
            pltpu.VMEM_SHARED((n_pad,), jnp.float32),
            pltpu.SemaphoreType.DMA,
        ],
    )
    def sc_deg(dst_hbm, out_hbm, dst_v, ones_v, acc, sem):
        c = lax.axis_index("c")
        s = lax.axis_index("s")
        w = c * _NS + s
        pltpu.sync_copy(dst_hbm.at[w], dst_v)
        for j in range(_K // 16):
            ones_v[pl.ds(j * 16, 16)] = jnp.zeros((16,), jnp.float32)
        for i in range(zchunks):
            pltpu.sync_copy(ones_v, acc.at[pl.ds(s * (n_pad // _NS) + i * _K, _K)])
        for j in range(_K // 16):
            ones_v[pl.ds(j * 16, 16)] = jnp.ones((16,), jnp.float32)
        plsc.subcore_barrier()

        def body(j, carry):
            pltpu.sync_copy(ones_v, acc.at[dst_v.at[j]], add=True)
            return carry

        lax.fori_loop(0, chunks, body, 0)
        plsc.subcore_barrier()
        rows = n_pad // _NS
        pltpu.sync_copy(acc.at[pl.ds(s * rows, rows)],
                        out_hbm.at[c, pl.ds(s * rows, rows)])

    return sc_deg


def _make_sc_scatter(n, n_pad, chunks, feat):
    zchunks = n_pad // _NS // _K
    out_rows = n // _NS

    @functools.partial(
        pl.kernel,
        mesh=_sc_mesh(),
        out_type=jax.ShapeDtypeStruct((_NC, n, feat), jnp.float32),
        scratch_types=[
            pltpu.VMEM((chunks, _K), jnp.int32),
            pltpu.VMEM((chunks, _K), jnp.int32),
            pltpu.VMEM((_K, feat), jnp.float32),
            pltpu.VMEM_SHARED((n_pad, feat), jnp.float32),
            pltpu.SemaphoreType.DMA,
        ],
    )
    def sc_scatter(src_hbm, dst_hbm, g_hbm, out_hbm, src_v, dst_v, rows_v, acc, sem):
        c = lax.axis_index("c")
        s = lax.axis_index("s")
        w = c * _NS + s
        pltpu.sync_copy(src_hbm.at[w], src_v)
        pltpu.sync_copy(dst_hbm.at[w], dst_v)

        def zrow(i, carry):
            for j in range(feat // 16):
                rows_v[i, pl.ds(j * 16, 16)] = jnp.zeros((16,), jnp.float32)
            return carry

        lax.fori_loop(0, _K, zrow, 0)
        for i in range(zchunks):
            pltpu.sync_copy(rows_v, acc.at[pl.ds(s * (n_pad // _NS) + i * _K, _K)])
        plsc.subcore_barrier()

        def body(j, carry):
            pltpu.async_copy(g_hbm.at[src_v.at[j]], rows_v, sem).wait()
            pltpu.sync_copy(rows_v, acc.at[dst_v.at[j]], add=True)
            return carry

        lax.fori_loop(0, chunks, body, 0)
        plsc.subcore_barrier()
        pltpu.sync_copy(acc.at[pl.ds(s * out_rows, out_rows)],
                        out_hbm.at[c, pl.ds(s * out_rows, out_rows)])

    return sc_scatter


def _row_spec(feat):
    return pl.BlockSpec((_BM, feat), lambda i: (i, 0))


def _col_spec():
    return pl.BlockSpec((_BM, 1), lambda i: (i, 0))


def _full_spec(shape):
    nd = len(shape)
    return pl.BlockSpec(shape, lambda i, _nd=nd: (0,) * _nd)


def _acc_spec(feat):
    return pl.BlockSpec((_NC, _BM, feat), lambda i: (0, i, 0))


def _tc_first(n, feat, delta):
    def body(x_ref, wp_ref, bp_ref, w0_ref, nb_ref, d0_ref, d1_ref,
             h0_ref, g0_ref, dinv_ref):
        deg = d0_ref[...] + d1_ref[...] + 1.0
        dinv = lax.rsqrt(deg)
        h0 = jnp.dot(x_ref[...], wp_ref[...],
                     preferred_element_type=jnp.float32) + bp_ref[...]
        nb = nb_ref[...]
        q = jnp.floor((h0 + nb) * (1.0 / delta)) * delta - nb
        g0 = dinv * jnp.dot(q, w0_ref[...], preferred_element_type=jnp.float32)
        h0_ref[...] = h0
        g0_ref[...] = g0
        dinv_ref[...] = dinv

    return pl.pallas_call(
        body,
        grid=(n // _BM,),
        in_specs=[
            _row_spec(feat), _full_spec((feat, feat)), _full_spec((1, feat)),
            _full_spec((feat, feat)), _row_spec(feat),
            _col_spec(), _col_spec(),
        ],
        out_specs=[_row_spec(feat), _row_spec(feat), _col_spec()],
        out_shape=[
            jax.ShapeDtypeStruct((n, feat), jnp.float32),
            jax.ShapeDtypeStruct((n, feat), jnp.float32),
            jax.ShapeDtypeStruct((n, 1), jnp.float32),
        ],
    )


def _tc_mid(n, feat, delta):
    def body(ap_ref, g_ref, dinv_ref, b_ref, eps_ref, nb_ref, w_ref, gout_ref):
        a = ap_ref[0] + ap_ref[1] + g_ref[...]
        dinv = dinv_ref[...]
        h = jnp.maximum(dinv * a + b_ref[...] + eps_ref[...], 0.0)
        nb = nb_ref[...]
        q = jnp.floor((h + nb) * (1.0 / delta)) * delta - nb
        gout_ref[...] = dinv * jnp.dot(q, w_ref[...],
                                       preferred_element_type=jnp.float32)

    return pl.pallas_call(
        body,
        grid=(n // _BM,),
        in_specs=[
            _acc_spec(feat), _row_spec(feat), _col_spec(),
            _full_spec((1, feat)), _row_spec(feat), _row_spec(feat),
            _full_spec((feat, feat)),
        ],
        out_specs=_row_spec(feat),
        out_shape=jax.ShapeDtypeStruct((n, feat), jnp.float32),
    )


def _tc_final(n, feat, out_c):
    def body(ap_ref, g_ref, dinv_ref, b_ref, eps_ref, h0_ref,
             wo1_ref, wo2_ref, bo_ref, out_ref):
        a = ap_ref[0] + ap_ref[1] + g_ref[...]
        h = jnp.maximum(dinv_ref[...] * a + b_ref[...] + eps_ref[...], 0.0)
        out_ref[...] = (
            jnp.dot(h0_ref[...], wo1_ref[...], preferred_element_type=jnp.float32)
            + jnp.dot(h, wo2_ref[...], preferred_element_type=jnp.float32)
            + bo_ref[...])

    return pl.pallas_call(
        body,
        grid=(n // _BM,),
        in_specs=[
            _acc_spec(feat), _row_spec(feat), _col_spec(),
            _full_spec((1, feat)), _row_spec(feat), _row_spec(feat),
            _full_spec((feat, out_c)), _full_spec((feat, out_c)),
            _full_spec((1, out_c)),
        ],
        out_specs=pl.BlockSpec((_BM, out_c), lambda i: (i, 0)),
        out_shape=jax.ShapeDtypeStruct((n, out_c), jnp.float32),
    )


def kernel(x, edge_index, W_proj, b_proj, Ws, bs, W_out, b_out):
    n, in_c = x.shape
    hid = W_proj.shape[1]
    num_layers = Ws.shape[0]
    out_c = W_out.shape[1]
    e = edge_index.shape[1]

    n_pad = -(-(n + 1) // (_NS * _K)) * (_NS * _K)
    chunks = -(-e // (_NW * _K))
    e_pad = _NW * chunks * _K

    src = edge_index[0]
    dst = edge_index[1]
    pad = e_pad - e
    src3 = jnp.concatenate([src, jnp.zeros((pad,), jnp.int32)]).reshape(
        _NW, chunks, _K)
    dst3 = jnp.concatenate([dst, jnp.full((pad,), n, jnp.int32)]).reshape(
        _NW, chunks, _K)

    deltas = [1.0 / 2 ** k for k in range(num_layers)]
    nkey = jax.random.key(42)
    nbs, epss = [], []
    for k in range(num_layers):
        kq = jax.random.fold_in(nkey, 2 * k)
        ke = jax.random.fold_in(nkey, 2 * k + 1)
        nbs.append((jax.random.uniform(kq, (n, hid), jnp.float32) - 0.5)
                   * deltas[k])
        epss.append(jax.random.normal(ke, (n, hid), jnp.float32) * 0.01)

    degp = _make_sc_deg(n_pad, chunks)(dst3)
    d0 = degp[0, :n].reshape(n, 1)
    d1 = degp[1, :n].reshape(n, 1)

    h0, g, dinv = _tc_first(n, hid, deltas[0])(
        x, W_proj, b_proj.reshape(1, hid), Ws[0], nbs[0], d0, d1)

    sc_scat = _make_sc_scatter(n, n_pad, chunks, hid)
    for k in range(1, num_layers):
        accp = sc_scat(src3, dst3, g)
        g = _tc_mid(n, hid, deltas[k])(
            accp, g, dinv, bs[k - 1].reshape(1, hid), epss[k - 1],
            nbs[k], Ws[k])

    accp = sc_scat(src3, dst3, g)
    out = _tc_final(n, hid, out_c)(
        accp, g, dinv, bs[num_layers - 1].reshape(1, hid),
        epss[num_layers - 1], h0, W_out[:hid], W_out[hid:],
        b_out.reshape(1, out_c))
    return out


# speedup vs baseline: 6.9655x; 6.9655x over previous
import functools

import jax
import jax.numpy as jnp
from jax import lax
from jax.experimental import pallas as pl
from jax.experimental.pallas import tpu as pltpu
from jax.experimental.pallas import tpu_sc as plsc

_NC = 2
_NS = 16
_NW = _NC * _NS
_K = 128
_NH = 2
_BM = 400


def _sc_mesh():
    return plsc.VectorSubcoreMesh(core_axis_name="c", subcore_axis_name="s",
                                  num_cores=_NC, num_subcores=_NS)


def _make_sc_deg(n_pad, ch):
    zfull, zrem = divmod(n_pad // _NS, _K)

    @functools.partial(
        pl.kernel,
        mesh=_sc_mesh(),
        out_type=jax.ShapeDtypeStruct((_NC * n_pad,), jnp.float32),
        scratch_types=[
            pltpu.VMEM((ch, _K), jnp.int32),
            pltpu.VMEM((_K,), jnp.float32),
            pltpu.VMEM_SHARED((n_pad,), jnp.float32),
            pltpu.SemaphoreType.DMA,
        ],
    )
    def sc_deg(dst_hbm, out_hbm, dst_v, ones_v, acc, sem):
        c = lax.axis_index("c")
        s = lax.axis_index("s")
        w = c * _NS + s
        for j in range(_K // 16):
            ones_v[pl.ds(j * 16, 16)] = jnp.zeros((16,), jnp.float32)
        for i in range(zfull):
            pltpu.sync_copy(ones_v, acc.at[pl.ds(s * (n_pad // _NS) + i * _K, _K)])
        if zrem:
            pltpu.sync_copy(ones_v.at[pl.ds(0, zrem)],
                            acc.at[pl.ds(s * (n_pad // _NS) + zfull * _K, zrem)])
        for j in range(_K // 16):
            ones_v[pl.ds(j * 16, 16)] = jnp.ones((16,), jnp.float32)
        plsc.subcore_barrier()

        def body(j, carry):
            pltpu.sync_copy(ones_v, acc.at[dst_v.at[j]], add=True)
            return carry

        for h in range(_NH):
            pltpu.sync_copy(dst_hbm.at[w, h], dst_v)
            lax.fori_loop(0, ch, body, 0)
        plsc.subcore_barrier()
        rows = n_pad // _NS
        pltpu.sync_copy(acc.at[pl.ds(s * rows, rows)],
                        out_hbm.at[pl.ds(c * n_pad + s * rows, rows)])

    return sc_deg


def _make_sc_scatter(n, n_pad, ch, feat):
    zfull, zrem = divmod(n_pad // _NS, _K)
    out_rows = n_pad // _NS

    @functools.partial(
        pl.kernel,
        mesh=_sc_mesh(),
        out_type=jax.ShapeDtypeStruct((_NC, n_pad, feat), jnp.float32),
        scratch_types=[
            pltpu.VMEM((ch, _K), jnp.int32),
            pltpu.VMEM((ch, _K), jnp.int32),
            pltpu.VMEM((_K, feat), jnp.float32),
            pltpu.VMEM((_K, feat), jnp.float32),
            pltpu.VMEM_SHARED((n_pad, feat), jnp.float32),
            pltpu.SemaphoreType.DMA,
            pltpu.SemaphoreType.DMA,
        ],
    )
    def sc_scatter(src_hbm, dst_hbm, g_hbm, out_hbm, src_v, dst_v,
                   rows_a, rows_b, acc, sem_a, sem_b):
        c = lax.axis_index("c")
        s = lax.axis_index("s")
        w = c * _NS + s

        def zrow(i, carry):
            for j in range(feat // 16):
                rows_a[i, pl.ds(j * 16, 16)] = jnp.zeros((16,), jnp.float32)
            return carry

        lax.fori_loop(0, _K, zrow, 0)
        for i in range(zfull):
            pltpu.sync_copy(rows_a, acc.at[pl.ds(s * (n_pad // _NS) + i * _K, _K)])
        if zrem:
            pltpu.sync_copy(rows_a.at[pl.ds(0, zrem)],
                            acc.at[pl.ds(s * (n_pad // _NS) + zfull * _K, zrem)])
        plsc.subcore_barrier()

        def body(jj, carry):
            j = 2 * jj
            pltpu.make_async_copy(g_hbm.at[src_v.at[j]], rows_a, sem_a).wait()
            pltpu.async_copy(g_hbm.at[src_v.at[j + 1]], rows_b, sem_b)
            pltpu.sync_copy(rows_a, acc.at[dst_v.at[j]], add=True)
            pltpu.make_async_copy(g_hbm.at[src_v.at[j + 1]], rows_b, sem_b).wait()
            pltpu.async_copy(g_hbm.at[src_v.at[j + 2]], rows_a, sem_a)
            pltpu.sync_copy(rows_b, acc.at[dst_v.at[j + 1]], add=True)
            return carry

        for h in range(_NH):
            pltpu.sync_copy(src_hbm.at[w, h], src_v)
            pltpu.sync_copy(dst_hbm.at[w, h], dst_v)
            pltpu.async_copy(g_hbm.at[src_v.at[0]], rows_a, sem_a)
            lax.fori_loop(0, ch // 2 - 1, body, 0)
            j = ch - 2
            pltpu.make_async_copy(g_hbm.at[src_v.at[j]], rows_a, sem_a).wait()
            pltpu.async_copy(g_hbm.at[src_v.at[j + 1]], rows_b, sem_b)
            pltpu.sync_copy(rows_a, acc.at[dst_v.at[j]], add=True)
            pltpu.make_async_copy(g_hbm.at[src_v.at[j + 1]], rows_b, sem_b).wait()
            pltpu.sync_copy(rows_b, acc.at[dst_v.at[j + 1]], add=True)
        plsc.subcore_barrier()
        pltpu.sync_copy(acc.at[pl.ds(s * out_rows, out_rows)],
                        out_hbm.at[c, pl.ds(s * out_rows, out_rows)])

    return sc_scatter


def _row_spec(feat):
    return pl.BlockSpec((_BM, feat), lambda i: (i, 0))


def _col_spec():
    return pl.BlockSpec((_BM, 1), lambda i: (i, 0))


def _full_spec(shape):
    nd = len(shape)
    return pl.BlockSpec(shape, lambda i, _nd=nd: (0,) * _nd)


def _acc_spec(feat):
    return pl.BlockSpec((_NC, _BM, feat), lambda i: (0, i, 0))


def _tc_first(n, feat, delta):
    def body(x_ref, wp_ref, bp_ref, w0_ref, nb_ref, d0_ref, d1_ref,
             h0_ref, g0_ref, dinv_ref):
        deg = d0_ref[...] + d1_ref[...] + 1.0
        dinv = lax.rsqrt(deg)
        h0 = jnp.dot(x_ref[...], wp_ref[...],
                     preferred_element_type=jnp.float32) + bp_ref[...]
        nb = nb_ref[...]
        q = jnp.floor((h0 + nb) * (1.0 / delta)) * delta - nb
        g0 = dinv * jnp.dot(q, w0_ref[...], preferred_element_type=jnp.float32)
        h0_ref[...] = h0
        g0_ref[...] = g0
        dinv_ref[...] = dinv

    return pl.pallas_call(
        body,
        grid=(n // _BM,),
        in_specs=[
            _row_spec(feat), _full_spec((feat, feat)), _full_spec((1, feat)),
            _full_spec((feat, feat)), _row_spec(feat),
            _col_spec(), _col_spec(),
        ],
        out_specs=[_row_spec(feat), _row_spec(feat), _col_spec()],
        out_shape=[
            jax.ShapeDtypeStruct((n, feat), jnp.float32),
            jax.ShapeDtypeStruct((n, feat), jnp.float32),
            jax.ShapeDtypeStruct((n, 1), jnp.float32),
        ],
    )


def _tc_mid(n, feat, delta):
    def body(ap_ref, g_ref, dinv_ref, b_ref, eps_ref, nb_ref, w_ref, gout_ref):
        a = ap_ref[0] + ap_ref[1] + g_ref[...]
        dinv = dinv_ref[...]
        h = jnp.maximum(dinv * a + b_ref[...] + eps_ref[...], 0.0)
        nb = nb_ref[...]
        q = jnp.floor((h + nb) * (1.0 / delta)) * delta - nb
        gout_ref[...] = dinv * jnp.dot(q, w_ref[...],
                                       preferred_element_type=jnp.float32)

    return pl.pallas_call(
        body,
        grid=(n // _BM,),
        in_specs=[
            _acc_spec(feat), _row_spec(feat), _col_spec(),
            _full_spec((1, feat)), _row_spec(feat), _row_spec(feat),
            _full_spec((feat, feat)),
        ],
        out_specs=_row_spec(feat),
        out_shape=jax.ShapeDtypeStruct((n, feat), jnp.float32),
    )


def _tc_final(n, feat, out_c):
    def body(ap_ref, g_ref, dinv_ref, b_ref, eps_ref, h0_ref,
             wo1_ref, wo2_ref, bo_ref, out_ref):
        a = ap_ref[0] + ap_ref[1] + g_ref[...]
        h = jnp.maximum(dinv_ref[...] * a + b_ref[...] + eps_ref[...], 0.0)
        out_ref[...] = (
            jnp.dot(h0_ref[...], wo1_ref[...], preferred_element_type=jnp.float32)
            + jnp.dot(h, wo2_ref[...], preferred_element_type=jnp.float32)
            + bo_ref[...])

    return pl.pallas_call(
        body,
        grid=(n // _BM,),
        in_specs=[
            _acc_spec(feat), _row_spec(feat), _col_spec(),
            _full_spec((1, feat)), _row_spec(feat), _row_spec(feat),
            _full_spec((feat, out_c)), _full_spec((feat, out_c)),
            _full_spec((1, out_c)),
        ],
        out_specs=pl.BlockSpec((_BM, out_c), lambda i: (i, 0)),
        out_shape=jax.ShapeDtypeStruct((n, out_c), jnp.float32),
    )


def kernel(x, edge_index, W_proj, b_proj, Ws, bs, W_out, b_out):
    n, in_c = x.shape
    hid = W_proj.shape[1]
    num_layers = Ws.shape[0]
    out_c = W_out.shape[1]
    e = edge_index.shape[1]

    n_pad = -(-(n + 1) // (_NS * 128)) * (_NS * 128)
    ch = -(-e // (_NW * _NH * _K))
    ch += ch % 2
    e_pad = _NW * _NH * ch * _K

    src = edge_index[0]
    dst = edge_index[1]
    pad = e_pad - e
    src3 = jnp.concatenate([src, jnp.zeros((pad,), jnp.int32)]).reshape(
        _NW, _NH, ch, _K)
    dst3 = jnp.concatenate([dst, jnp.full((pad,), n, jnp.int32)]).reshape(
        _NW, _NH, ch, _K)

    deltas = [1.0 / 2 ** k for k in range(num_layers)]
    nkey = jax.random.key(42)
    nbs, epss = [], []
    for k in range(num_layers):
        kq = jax.random.fold_in(nkey, 2 * k)
        ke = jax.random.fold_in(nkey, 2 * k + 1)
        nbs.append((jax.random.uniform(kq, (n, hid), jnp.float32) - 0.5)
                   * deltas[k])
        epss.append(jax.random.normal(ke, (n, hid), jnp.float32) * 0.01)

    degp = _make_sc_deg(n_pad, ch)(dst3)
    d0 = degp[:n].reshape(n, 1)
    d1 = degp[n_pad:n_pad + n].reshape(n, 1)

    h0, g, dinv = _tc_first(n, hid, deltas[0])(
        x, W_proj, b_proj.reshape(1, hid), Ws[0], nbs[0], d0, d1)

    sc_scat = _make_sc_scatter(n, n_pad, ch, hid)
    for k in range(1, num_layers):
        accp = sc_scat(src3, dst3, g)
        g = _tc_mid(n, hid, deltas[k])(
            accp, g, dinv, bs[k - 1].reshape(1, hid), epss[k - 1],
            nbs[k], Ws[k])

    accp = sc_scat(src3, dst3, g)
    out = _tc_final(n, hid, out_c)(
        accp, g, dinv, bs[num_layers - 1].reshape(1, hid),
        epss[num_layers - 1], h0, W_out[:hid], W_out[hid:],
        b_out.reshape(1, out_c))
    return out
